# baseline (speedup 1.0000x reference)
import jax
import jax.numpy as jnp
from jax.experimental import pallas as pl
from jax.experimental.pallas import tpu as pltpu

_BI = 2000
_BK = 1024


def _sage_kernel(adj_ref, xa_ref, xr_ref, w_ref, b_ref, out_ref,
                 acc_ref, *, d_in):
    k = pl.program_id(1)
    nk = pl.num_programs(1)

    @pl.when(k == 0)
    def _init():
        acc_ref[...] = jnp.zeros_like(acc_ref)

    a = adj_ref[...].astype(jnp.bfloat16)
    acc_ref[...] += jnp.dot(a, xa_ref[...], preferred_element_type=jnp.float32)

    @pl.when(k == nk - 1)
    def _epilogue():
        acc = acc_ref[...]
        x1 = acc[:, :d_in] / acc[:, d_in:d_in + 1]
        w = w_ref[...]
        out_ref[...] = (
            jnp.dot(x1, w[:d_in], preferred_element_type=jnp.float32)
            + jnp.dot(xr_ref[...], w[d_in:], preferred_element_type=jnp.float32)
            + b_ref[...]
        )


def kernel(x, adj, weight, bias):
    n, d_in = x.shape
    d_out = weight.shape[1]
    nk = pl.cdiv(n, _BK)
    npad = nk * _BK
    xa = jnp.zeros((npad, 2 * d_in), jnp.bfloat16)
    xa = xa.at[:n, :d_in].set(x.astype(jnp.bfloat16))
    xa = xa.at[:n, d_in].set(jnp.float32(1.0).astype(jnp.bfloat16))
    bias2d = bias.reshape(1, d_out)

    grid = (n // _BI, nk)
    out = pl.pallas_call(
        lambda *refs: _sage_kernel(*refs, d_in=d_in),
        grid=grid,
        in_specs=[
            pl.BlockSpec((_BI, _BK), lambda i, k: (i, k)),
            pl.BlockSpec((_BK, 2 * d_in), lambda i, k: (k, 0)),
            pl.BlockSpec((_BI, d_in), lambda i, k: (i, 0)),
            pl.BlockSpec((2 * d_in, d_out), lambda i, k: (0, 0)),
            pl.BlockSpec((1, d_out), lambda i, k: (0, 0)),
        ],
        out_specs=pl.BlockSpec((_BI, d_out), lambda i, k: (i, 0)),
        out_shape=jax.ShapeDtypeStruct((n, d_out), jnp.float32),
        scratch_shapes=[
            pltpu.VMEM((_BI, 2 * d_in), jnp.float32),
        ],
        compiler_params=pltpu.CompilerParams(
            dimension_semantics=("parallel", "arbitrary"),
        ),
    )(adj, xa, x, weight, bias2d)
    return out
